# baseline (speedup 1.0000x reference)
import functools

import jax
import jax.numpy as jnp
from jax import lax
from jax.experimental import pallas as pl
from jax.experimental.pallas import tpu as pltpu
from jax.experimental.pallas import tpu_sc as plsc

NUM_REC = 100
NUM_ITEMS = 10000
EMBED_DIM = 32
HIST_LEN = 200

NUM_WORKERS = 32
N_PAD = 10240
ROWS_PER_W = N_PAD // NUM_WORKERS
HALF = EMBED_DIM // 2

_sc_mesh = plsc.VectorSubcoreMesh(core_axis_name="c", subcore_axis_name="s")


@functools.partial(
    pl.kernel,
    mesh=_sc_mesh,
    out_type=jax.ShapeDtypeStruct((N_PAD,), jnp.float32),
    scratch_types=[
        pltpu.VMEM((HIST_LEN,), jnp.int32),
        pltpu.VMEM((HIST_LEN, EMBED_DIM), jnp.float32),
        pltpu.VMEM((ROWS_PER_W, EMBED_DIM), jnp.float32),
        pltpu.VMEM((ROWS_PER_W,), jnp.float32),
        pltpu.SemaphoreType.DMA,
    ],
    compiler_params=pltpu.CompilerParams(
        needs_layout_passes=False, use_tc_tiling_on_sc=False
    ),
)
def _sc_scores(hist_hbm, table_hbm, out_hbm, idx_v, rows_v, chunk_v, sc_v, sem):
    wid = lax.axis_index("s") * 2 + lax.axis_index("c")
    base = jnp.minimum(wid * ROWS_PER_W, NUM_ITEMS - ROWS_PER_W)

    chunk_cp = pltpu.make_async_copy(
        table_hbm.at[pl.ds(base, ROWS_PER_W)], chunk_v, sem
    )
    chunk_cp.start()
    pltpu.sync_copy(hist_hbm, idx_v)
    pltpu.async_copy(table_hbm.at[idx_v], rows_v, sem).wait()
    chunk_cp.wait()

    zero = jnp.zeros((HALF,), jnp.float32)

    def mean_body(i, carry):
        a0, a1 = carry
        return (a0 + rows_v[i, pl.ds(0, HALF)], a1 + rows_v[i, pl.ds(HALF, HALF)])

    a0, a1 = lax.fori_loop(0, HIST_LEN, mean_body, (zero, zero))
    scale = jnp.float32(1.0 / HIST_LEN)

    def bf16q(x):
        b = lax.bitcast_convert_type(x, jnp.int32)
        lsb = lax.shift_right_logical(b, 16) & 1
        b = (b + (0x7FFF + lsb)) & jnp.int32(-65536)
        return lax.bitcast_convert_type(b, jnp.float32)

    u0 = bf16q(a0 * scale)
    u1 = bf16q(a1 * scale)

    lane = lax.iota(jnp.int32, HALF)

    def group_body(g, _):
        acc = jnp.zeros((HALF,), jnp.float32)
        for i in range(HALF):
            r = g * HALF + i
            v = bf16q(chunk_v[r, pl.ds(0, HALF)]) * u0 + bf16q(
                chunk_v[r, pl.ds(HALF, HALF)]
            ) * u1
            acc = jnp.where(lane == i, jnp.sum(v), acc)
        sc_v[pl.ds(g * HALF, HALF)] = acc
        return 0

    lax.fori_loop(0, ROWS_PER_W // HALF, group_body, 0)
    pltpu.sync_copy(sc_v, out_hbm.at[pl.ds(base, ROWS_PER_W)])


def _allreduce(x, op):
    for sh in (1, 2, 4):
        x = op(x, pltpu.roll(x, sh, 0))
    for sh in (1, 2, 4, 8, 16, 32, 64):
        x = op(x, pltpu.roll(x, sh, 1))
    return x


def _tc_topk_body(s_ref, out_ref):
    rows = N_PAD // 128
    ngrp = rows // 8
    s = s_ref[...].reshape(rows, 128)
    lin = (
        lax.broadcasted_iota(jnp.int32, (rows, 128), 0) * 128
        + lax.broadcasted_iota(jnp.int32, (rows, 128), 1)
    )
    s = jnp.where(lin < NUM_ITEMS, s, -jnp.inf)
    lin8 = (
        lax.broadcasted_iota(jnp.int32, (8, 128), 0) * 128
        + lax.broadcasted_iota(jnp.int32, (8, 128), 1)
    )

    def remax(s):
        vals = [s[g * 8 : (g + 1) * 8, :] for g in range(ngrp)]
        gs = [jnp.full((8, 128), g, jnp.int32) for g in range(ngrp)]
        while len(vals) > 1:
            nv, ng = [], []
            for i in range(0, len(vals) - 1, 2):
                upd = vals[i + 1] > vals[i]
                nv.append(jnp.where(upd, vals[i + 1], vals[i]))
                ng.append(jnp.where(upd, gs[i + 1], gs[i]))
            if len(vals) % 2:
                nv.append(vals[-1])
                ng.append(gs[-1])
            vals, gs = nv, ng
        return vals[0], gs[0]

    def body(k, carry):
        s, val8, g8, acc8 = carry
        m8 = _allreduce(val8, jnp.maximum)
        cand = jnp.where(val8 == m8, g8 * 1024 + lin8, jnp.int32(2**30))
        pos8 = _allreduce(cand, jnp.minimum)
        acc8 = jnp.where(lin8 == k, pos8, acc8)
        s = jnp.where(lin == pos8[0:1, :], -jnp.inf, s)
        val8, g8 = remax(s)
        return s, val8, g8, acc8

    v0, g0 = remax(s)
    acc0 = jnp.zeros((8, 128), jnp.int32)
    _, _, _, acc8 = lax.fori_loop(0, NUM_REC, body, (s, v0, g0, acc0))
    out_ref[...] = acc8


def _tc_topk(scores):
    return pl.pallas_call(
        _tc_topk_body,
        in_specs=[pl.BlockSpec(memory_space=pltpu.VMEM)],
        out_specs=pl.BlockSpec(memory_space=pltpu.VMEM),
        out_shape=jax.ShapeDtypeStruct((8, 128), jnp.int32),
    )(scores)


@jax.jit
def kernel(user_history, item_embeddings):
    hist = user_history.astype(jnp.int32)
    scores = _sc_scores(hist, item_embeddings)
    return _tc_topk(scores).reshape(1024)[:NUM_REC]


# speedup vs baseline: 2.0692x; 2.0692x over previous
import functools

import jax
import jax.numpy as jnp
from jax import lax
from jax.experimental import pallas as pl
from jax.experimental.pallas import tpu as pltpu
from jax.experimental.pallas import tpu_sc as plsc

NUM_REC = 100
NUM_ITEMS = 10000
EMBED_DIM = 32
HIST_LEN = 200

NUM_WORKERS = 32
N_PAD = 10240
ROWS_PER_W = N_PAD // NUM_WORKERS
HALF = EMBED_DIM // 2

_sc_mesh = plsc.VectorSubcoreMesh(core_axis_name="c", subcore_axis_name="s")


@functools.partial(
    pl.kernel,
    mesh=_sc_mesh,
    out_type=jax.ShapeDtypeStruct((N_PAD,), jnp.float32),
    scratch_types=[
        pltpu.VMEM((HIST_LEN,), jnp.int32),
        pltpu.VMEM((HIST_LEN, EMBED_DIM), jnp.float32),
        pltpu.VMEM((ROWS_PER_W, EMBED_DIM), jnp.float32),
        pltpu.VMEM((ROWS_PER_W,), jnp.float32),
        pltpu.SemaphoreType.DMA,
    ],
    compiler_params=pltpu.CompilerParams(
        needs_layout_passes=False, use_tc_tiling_on_sc=False
    ),
)
def _sc_scores(hist_hbm, table_hbm, out_hbm, idx_v, rows_v, chunk_v, sc_v, sem):
    wid = lax.axis_index("s") * 2 + lax.axis_index("c")
    base = jnp.minimum(wid * ROWS_PER_W, NUM_ITEMS - ROWS_PER_W)

    chunk_cp = pltpu.make_async_copy(
        table_hbm.at[pl.ds(base, ROWS_PER_W)], chunk_v, sem
    )
    chunk_cp.start()
    pltpu.sync_copy(hist_hbm, idx_v)
    pltpu.async_copy(table_hbm.at[idx_v], rows_v, sem).wait()
    chunk_cp.wait()

    zero = jnp.zeros((HALF,), jnp.float32)

    def mean_body(i, carry):
        a0, a1 = carry
        return (a0 + rows_v[i, pl.ds(0, HALF)], a1 + rows_v[i, pl.ds(HALF, HALF)])

    a0, a1 = lax.fori_loop(0, HIST_LEN, mean_body, (zero, zero))
    scale = jnp.float32(1.0 / HIST_LEN)

    def bf16q(x):
        b = lax.bitcast_convert_type(x, jnp.int32)
        lsb = lax.shift_right_logical(b, 16) & 1
        b = (b + (0x7FFF + lsb)) & jnp.int32(-65536)
        return lax.bitcast_convert_type(b, jnp.float32)

    u0 = bf16q(a0 * scale)
    u1 = bf16q(a1 * scale)

    lane = lax.iota(jnp.int32, HALF)

    def group_body(g, _):
        acc = jnp.zeros((HALF,), jnp.float32)
        for i in range(HALF):
            r = g * HALF + i
            v = bf16q(chunk_v[r, pl.ds(0, HALF)]) * u0 + bf16q(
                chunk_v[r, pl.ds(HALF, HALF)]
            ) * u1
            acc = jnp.where(lane == i, jnp.sum(v), acc)
        sc_v[pl.ds(g * HALF, HALF)] = acc
        return 0

    lax.fori_loop(0, ROWS_PER_W // HALF, group_body, 0)
    pltpu.sync_copy(sc_v, out_hbm.at[pl.ds(base, ROWS_PER_W)])


def _tc_topk_body(s_ref, out_ref):
    rows = N_PAD // 128
    ngrp = rows // 8
    s = s_ref[...].reshape(rows, 128)
    lin = (
        lax.broadcasted_iota(jnp.int32, (rows, 128), 0) * 128
        + lax.broadcasted_iota(jnp.int32, (rows, 128), 1)
    )
    s = jnp.where(lin < NUM_ITEMS, s, -jnp.inf)
    lin8 = (
        lax.broadcasted_iota(jnp.int32, (8, 128), 0) * 128
        + lax.broadcasted_iota(jnp.int32, (8, 128), 1)
    )

    def remax(s):
        vals = [s[g * 8 : (g + 1) * 8, :] for g in range(ngrp)]
        gs = [jnp.full((8, 128), g, jnp.int32) for g in range(ngrp)]
        while len(vals) > 1:
            nv, ng = [], []
            for i in range(0, len(vals) - 1, 2):
                upd = vals[i + 1] > vals[i]
                nv.append(jnp.where(upd, vals[i + 1], vals[i]))
                ng.append(jnp.where(upd, gs[i + 1], gs[i]))
            if len(vals) % 2:
                nv.append(vals[-1])
                ng.append(gs[-1])
            vals, gs = nv, ng
        return vals[0], gs[0]

    def body(k, carry):
        s, val8, g8, acc8 = carry
        m = jnp.max(val8)
        pos = jnp.min(jnp.where(val8 == m, g8 * 1024 + lin8, jnp.int32(2**30)))
        acc8 = jnp.where(lin8 == k, pos, acc8)
        s = jnp.where(lin == pos, -jnp.inf, s)
        val8, g8 = remax(s)
        return s, val8, g8, acc8

    v0, g0 = remax(s)
    acc0 = jnp.zeros((8, 128), jnp.int32)
    _, _, _, acc8 = lax.fori_loop(0, NUM_REC, body, (s, v0, g0, acc0))
    out_ref[...] = acc8


def _tc_topk(scores):
    return pl.pallas_call(
        _tc_topk_body,
        in_specs=[pl.BlockSpec(memory_space=pltpu.VMEM)],
        out_specs=pl.BlockSpec(memory_space=pltpu.VMEM),
        out_shape=jax.ShapeDtypeStruct((8, 128), jnp.int32),
    )(scores)


@jax.jit
def kernel(user_history, item_embeddings):
    hist = user_history.astype(jnp.int32)
    scores = _sc_scores(hist, item_embeddings)
    return _tc_topk(scores).reshape(1024)[:NUM_REC]


# speedup vs baseline: 2.0694x; 1.0001x over previous
import functools

import jax
import jax.numpy as jnp
from jax import lax
from jax.experimental import pallas as pl
from jax.experimental.pallas import tpu as pltpu
from jax.experimental.pallas import tpu_sc as plsc

NUM_REC = 100
NUM_ITEMS = 10000
EMBED_DIM = 32
HIST_LEN = 200

NUM_WORKERS = 32
N_PAD = 10240
ROWS_PER_W = N_PAD // NUM_WORKERS
HALF = EMBED_DIM // 2

_sc_mesh = plsc.VectorSubcoreMesh(core_axis_name="c", subcore_axis_name="s")


@functools.partial(
    pl.kernel,
    mesh=_sc_mesh,
    out_type=jax.ShapeDtypeStruct((N_PAD,), jnp.float32),
    scratch_types=[
        pltpu.VMEM((HIST_LEN,), jnp.int32),
        pltpu.VMEM((HIST_LEN, EMBED_DIM), jnp.float32),
        pltpu.VMEM((ROWS_PER_W, EMBED_DIM), jnp.float32),
        pltpu.VMEM((ROWS_PER_W,), jnp.float32),
        pltpu.SemaphoreType.DMA,
    ],
    compiler_params=pltpu.CompilerParams(
        needs_layout_passes=False, use_tc_tiling_on_sc=False
    ),
)
def _sc_scores(hist_hbm, table_hbm, out_hbm, idx_v, rows_v, chunk_v, sc_v, sem):
    wid = lax.axis_index("s") * 2 + lax.axis_index("c")
    base = jnp.minimum(wid * ROWS_PER_W, NUM_ITEMS - ROWS_PER_W)

    chunk_cp = pltpu.make_async_copy(
        table_hbm.at[pl.ds(base, ROWS_PER_W)], chunk_v, sem
    )
    chunk_cp.start()
    pltpu.sync_copy(hist_hbm, idx_v)
    pltpu.async_copy(table_hbm.at[idx_v], rows_v, sem).wait()
    chunk_cp.wait()

    zero = jnp.zeros((HALF,), jnp.float32)

    def mean_body(i, carry):
        a0, a1 = carry
        return (a0 + rows_v[i, pl.ds(0, HALF)], a1 + rows_v[i, pl.ds(HALF, HALF)])

    a0, a1 = lax.fori_loop(0, HIST_LEN, mean_body, (zero, zero))
    scale = jnp.float32(1.0 / HIST_LEN)

    def bf16q(x):
        b = lax.bitcast_convert_type(x, jnp.int32)
        lsb = lax.shift_right_logical(b, 16) & 1
        b = (b + (0x7FFF + lsb)) & jnp.int32(-65536)
        return lax.bitcast_convert_type(b, jnp.float32)

    u0 = bf16q(a0 * scale)
    u1 = bf16q(a1 * scale)

    lane = lax.iota(jnp.int32, HALF)

    def group_body(g, _):
        acc = jnp.zeros((HALF,), jnp.float32)
        for i in range(HALF):
            r = g * HALF + i
            v = bf16q(chunk_v[r, pl.ds(0, HALF)]) * u0 + bf16q(
                chunk_v[r, pl.ds(HALF, HALF)]
            ) * u1
            acc = jnp.where(lane == i, jnp.sum(v), acc)
        sc_v[pl.ds(g * HALF, HALF)] = acc
        return 0

    lax.fori_loop(0, ROWS_PER_W // HALF, group_body, 0)
    pltpu.sync_copy(sc_v, out_hbm.at[pl.ds(base, ROWS_PER_W)])


def _tc_topk_body(s_ref, out_ref):
    rows = N_PAD // 128
    ngrp = rows // 8
    s = s_ref[...].reshape(rows, 128)
    lin = (
        lax.broadcasted_iota(jnp.int32, (rows, 128), 0) * 128
        + lax.broadcasted_iota(jnp.int32, (rows, 128), 1)
    )
    s = jnp.where(lin < NUM_ITEMS, s, -jnp.inf)
    lin8 = (
        lax.broadcasted_iota(jnp.int32, (8, 128), 0) * 128
        + lax.broadcasted_iota(jnp.int32, (8, 128), 1)
    )

    def remax(s):
        vals = [s[g * 8 : (g + 1) * 8, :] for g in range(ngrp)]
        gs = [jnp.full((8, 128), g, jnp.int32) for g in range(ngrp)]
        while len(vals) > 1:
            nv, ng = [], []
            for i in range(0, len(vals) - 1, 2):
                upd = vals[i + 1] > vals[i]
                nv.append(jnp.where(upd, vals[i + 1], vals[i]))
                ng.append(jnp.where(upd, gs[i + 1], gs[i]))
            if len(vals) % 2:
                nv.append(vals[-1])
                ng.append(gs[-1])
            vals, gs = nv, ng
        return vals[0], gs[0]

    def body(k, carry):
        s, val8, g8, acc8 = carry
        m = jnp.max(val8, keepdims=True)
        pos = jnp.min(
            jnp.where(val8 == m, g8 * 1024 + lin8, jnp.int32(2**30)), keepdims=True
        )
        acc8 = jnp.where(lin8 == k, pos, acc8)
        s = jnp.where(lin == pos, -jnp.inf, s)
        val8, g8 = remax(s)
        return s, val8, g8, acc8

    v0, g0 = remax(s)
    acc0 = jnp.zeros((8, 128), jnp.int32)
    _, _, _, acc8 = lax.fori_loop(0, NUM_REC, body, (s, v0, g0, acc0))
    out_ref[...] = acc8


def _tc_topk(scores):
    return pl.pallas_call(
        _tc_topk_body,
        in_specs=[pl.BlockSpec(memory_space=pltpu.VMEM)],
        out_specs=pl.BlockSpec(memory_space=pltpu.VMEM),
        out_shape=jax.ShapeDtypeStruct((8, 128), jnp.int32),
    )(scores)


@jax.jit
def kernel(user_history, item_embeddings):
    hist = user_history.astype(jnp.int32)
    scores = _sc_scores(hist, item_embeddings)
    return _tc_topk(scores).reshape(1024)[:NUM_REC]
